# baseline (speedup 1.0000x reference)
import functools

import jax
import jax.numpy as jnp
from jax import lax
from jax.experimental import pallas as pl
from jax.experimental.pallas import tpu as pltpu
from jax.experimental.pallas import tpu_sc as plsc

_R, _C = 64, 32768
_NC, _NS, _L = 2, 16, 16
_NW = _NC * _NS
_ROWS_W = _R // _NW
_CHUNK = 8192
_NCHUNK = _C // _CHUNK
_SB = 512
_WPR = _C // 32

_mesh = plsc.VectorSubcoreMesh(core_axis_name="c", subcore_axis_name="s")


@functools.partial(
    pl.kernel,
    out_type=jax.ShapeDtypeStruct((_R, _C), jnp.float32),
    mesh=_mesh,
    scratch_types=[
        pltpu.VMEM((_ROWS_W, _CHUNK), jnp.float32),
        pltpu.VMEM((_ROWS_W, _CHUNK // 32), jnp.int32),
    ],
)
def _domain_mask_sc(w_hbm, b_hbm, out_hbm, w_v, b_v):
    wid = lax.axis_index("s") * _NC + lax.axis_index("c")
    r0 = wid * _ROWS_W

    zero = jnp.zeros((_L,), jnp.float32)

    for c in range(_NCHUNK):
        col = c * _CHUNK
        pltpu.sync_copy(w_hbm.at[pl.ds(r0, _ROWS_W), pl.ds(col, _CHUNK)], w_v)
        pltpu.sync_copy(
            b_hbm.at[pl.ds(r0, _ROWS_W), pl.ds(col // 32, _CHUNK // 32)], b_v
        )

        for r in range(_ROWS_W):
            @plsc.parallel_loop(0, _CHUNK // _SB)
            def _body(p):
                words = b_v[r, pl.ds(p * _L, _L)]
                for q in range(32):
                    off = p * _SB + q * _L
                    vec = w_v[r, pl.ds(off, _L)]
                    hit = (words << (31 - q)) < 0
                    w_v[r, pl.ds(off, _L)] = jnp.where(hit, vec, zero)

        pltpu.sync_copy(w_v, out_hbm.at[pl.ds(r0, _ROWS_W), pl.ds(col, _CHUNK)])


def _pack_mask(mask):
    mb = mask.reshape(_R, _C // _SB, 32, _L).astype(jnp.uint32)
    weights = jnp.left_shift(
        jnp.uint32(1), jnp.arange(32, dtype=jnp.uint32)
    )[None, None, :, None]
    packed = (mb * weights).sum(axis=2, dtype=jnp.uint32)
    return lax.bitcast_convert_type(packed, jnp.int32).reshape(_R, _WPR)


def kernel(w, mask):
    return _domain_mask_sc(w, _pack_mask(mask))


# speedup vs baseline: 1.1689x; 1.1689x over previous
import functools

import jax
import jax.numpy as jnp
from jax import lax
from jax.experimental import pallas as pl
from jax.experimental.pallas import tpu as pltpu
from jax.experimental.pallas import tpu_sc as plsc

_R, _C = 64, 32768
_NC, _NS, _L = 2, 16, 16
_NW = _NC * _NS
_ROWS_W = _R // _NW
_CHUNK = 8192
_NCHUNK = _C // _CHUNK
_SB = 512
_WPR = _C // 32
_BCH = _CHUNK // 32

_mesh = plsc.VectorSubcoreMesh(core_axis_name="c", subcore_axis_name="s")


@functools.partial(
    pl.kernel,
    out_type=jax.ShapeDtypeStruct((_R, _C), jnp.float32),
    mesh=_mesh,
    scratch_types=[
        pltpu.VMEM((_NCHUNK, _ROWS_W, _CHUNK), jnp.float32),
        pltpu.VMEM((_NCHUNK, _ROWS_W, _BCH), jnp.int32),
        pltpu.SemaphoreType.DMA((_NCHUNK,)),
        pltpu.SemaphoreType.DMA((_NCHUNK,)),
        pltpu.SemaphoreType.DMA((_NCHUNK,)),
    ],
)
def _domain_mask_sc(w_hbm, b_hbm, out_hbm, w_v, b_v, s_w, s_b, s_o):
    wid = lax.axis_index("s") * _NC + lax.axis_index("c")
    r0 = wid * _ROWS_W

    zero = jnp.zeros((_L,), jnp.float32)
    rows = pl.ds(r0, _ROWS_W)

    in_w, in_b, out_h = [], [], []
    for c in range(_NCHUNK):
        col = c * _CHUNK
        in_w.append(pltpu.async_copy(
            w_hbm.at[rows, pl.ds(col, _CHUNK)], w_v.at[c], s_w.at[c]))
        in_b.append(pltpu.async_copy(
            b_hbm.at[rows, pl.ds(c * _BCH, _BCH)], b_v.at[c], s_b.at[c]))

    for c in range(_NCHUNK):
        in_w[c].wait()
        in_b[c].wait()

        for r in range(_ROWS_W):
            @plsc.parallel_loop(0, _CHUNK // _SB)
            def _body(p):
                words = b_v[c, r, pl.ds(p * _L, _L)]
                for q in range(32):
                    off = p * _SB + q * _L
                    vec = w_v[c, r, pl.ds(off, _L)]
                    hit = (words << (31 - q)) < 0
                    w_v[c, r, pl.ds(off, _L)] = jnp.where(hit, vec, zero)

        out_h.append(pltpu.async_copy(
            w_v.at[c], out_hbm.at[rows, pl.ds(c * _CHUNK, _CHUNK)], s_o.at[c]))

    for h in out_h:
        h.wait()


def _pack_mask(mask):
    mb = mask.reshape(_R, _C // _SB, 32, _L).astype(jnp.uint32)
    weights = jnp.left_shift(
        jnp.uint32(1), jnp.arange(32, dtype=jnp.uint32)
    )[None, None, :, None]
    packed = (mb * weights).sum(axis=2, dtype=jnp.uint32)
    return lax.bitcast_convert_type(packed, jnp.int32).reshape(_R, _WPR)


def kernel(w, mask):
    return _domain_mask_sc(w, _pack_mask(mask))
